# baseline (speedup 1.0000x reference)
import functools

import jax
import jax.numpy as jnp
from jax import lax
from jax.experimental import pallas as pl
from jax.experimental.pallas import tpu as pltpu
from jax.experimental.pallas import tpu_sc as plsc

N_TOKENS = 32768
D_MODEL = 2048
N_EXPERTS = 8
TOP_K = 2

NC = 2
NS = 16
L = 16
NW = NC * NS
CHUNK = N_TOKENS // NW
STEPS = CHUNK // L
BLK = 128

TOK_TILE = 1024
N_STEPS = N_TOKENS // TOK_TILE
NBUF = 5
LOOKAHEAD = 4


def _copy_slot(x_hbm, xbuf, sems, step, s):
    return pltpu.make_async_copy(
        x_hbm.at[pl.ds(step * TOK_TILE, TOK_TILE), :],
        xbuf.at[s],
        sems.at[s],
    )


def _out_copy(obuf, out_hbm, osems, step, s):
    return pltpu.make_async_copy(
        obuf.at[s],
        out_hbm.at[:, pl.ds(step * TOK_TILE, TOK_TILE)],
        osems.at[s],
    )


def _matmul_body(x_hbm, w_ref, out_hbm, xbuf, obuf, sems, osems):
    i = pl.program_id(0)

    def start(step):
        for s in range(NBUF):
            @pl.when(lax.rem(step, NBUF) == s)
            def _(s=s):
                _copy_slot(x_hbm, xbuf, sems, step, s).start()

    @pl.when(i == 0)
    def _():
        for d in range(LOOKAHEAD):
            _copy_slot(x_hbm, xbuf, sems, d, d).start()

    @pl.when(i + LOOKAHEAD < N_STEPS)
    def _():
        start(i + LOOKAHEAD)

    os_ = lax.rem(i, 2)
    for t in range(2):
        @pl.when((i >= 2) & (os_ == t))
        def _(t=t):
            _out_copy(obuf, out_hbm, osems, i - 2, t).wait()

    for s in range(NBUF):
        @pl.when(lax.rem(i, NBUF) == s)
        def _(s=s):
            _copy_slot(x_hbm, xbuf, sems, i, s).wait()
            for t in range(2):
                @pl.when(os_ == t)
                def _(s=s, t=t):
                    obuf[t] = lax.dot_general(
                        w_ref[...], xbuf[s],
                        dimension_numbers=(((0,), (1,)), ((), ())),
                        preferred_element_type=jnp.float32,
                    )
                    _out_copy(obuf, out_hbm, osems, i, t).start()

    @pl.when(i == N_STEPS - 1)
    def _():
        for t in range(2):
            @pl.when(os_ == t)
            def _(t=t):
                _out_copy(obuf, out_hbm, osems, i - 1, 1 - t).wait()
                _out_copy(obuf, out_hbm, osems, i, t).wait()


def _matmul(x, w):
    return pl.pallas_call(
        _matmul_body,
        grid=(N_STEPS,),
        in_specs=[
            pl.BlockSpec(memory_space=pl.ANY),
            pl.BlockSpec((D_MODEL, N_EXPERTS), lambda i: (0, 0)),
        ],
        out_specs=pl.BlockSpec(memory_space=pl.ANY),
        out_shape=jax.ShapeDtypeStruct((N_EXPERTS, N_TOKENS), jnp.float32),
        scratch_shapes=[
            pltpu.VMEM((NBUF, TOK_TILE, D_MODEL), jnp.float32),
            pltpu.VMEM((2, N_EXPERTS, TOK_TILE), jnp.float32),
            pltpu.SemaphoreType.DMA((NBUF,)),
            pltpu.SemaphoreType.DMA((2,)),
        ],
        compiler_params=pltpu.CompilerParams(
            dimension_semantics=("arbitrary",),
            vmem_limit_bytes=48 * 1024 * 1024,
        ),
    )(x, w)


def _router_body(logits_hbm, b_hbm, gates_hbm, scores_hbm, idxs_hbm,
                 le_v, b_v, g_v, s_v, i_v, dma_sem):
    wid = lax.axis_index("s") * NC + lax.axis_index("c")
    base = wid * CHUNK

    def le_copy(e):
        return pltpu.make_async_copy(
            logits_hbm.at[e, pl.ds(base, CHUNK)],
            le_v.at[pl.ds(e * CHUNK, CHUNK)],
            dma_sem,
        )

    for e in range(N_EXPERTS):
        le_copy(e).start()
    pltpu.sync_copy(b_hbm, b_v)
    for e in range(N_EXPERTS):
        le_copy(e).wait()
    bsp = [plsc.load_gather(b_v, [jnp.full((L,), e, jnp.int32)])
           for e in range(N_EXPERTS)]

    @plsc.parallel_loop(0, STEPS, 1, unroll=4)
    def step(j):
        off = j * L
        i1 = jnp.zeros((L,), jnp.int32)
        m1 = le_v[pl.ds(off, L)] + bsp[0]
        m2 = jnp.full((L,), -jnp.inf, jnp.float32)
        i2 = jnp.zeros((L,), jnp.int32)
        for e in range(1, N_EXPERTS):
            col = jnp.full((L,), e, jnp.int32)
            v = le_v[pl.ds(e * CHUNK + off, L)] + bsp[e]
            gt1 = v > m1
            gt2 = v > m2
            m2 = jnp.where(gt1, m1, jnp.where(gt2, v, m2))
            i2 = jnp.where(gt1, i1, jnp.where(gt2, col, i2))
            m1 = jnp.where(gt1, v, m1)
            i1 = jnp.where(gt1, col, i1)
        r = jnp.exp(m2 - m1)
        g1 = 1.0 / (1.0 + r)
        g2 = r * g1
        d0 = (j // (BLK // L)) * (TOP_K * BLK) + (j % (BLK // L)) * L
        d1 = d0 + BLK
        s_v[pl.ds(d0, L)] = m1
        s_v[pl.ds(d1, L)] = m2
        g_v[pl.ds(d0, L)] = g1
        g_v[pl.ds(d1, L)] = g2
        i_v[pl.ds(d0, L)] = i1
        i_v[pl.ds(d1, L)] = i2

    obase = base * TOP_K
    pltpu.sync_copy(g_v, gates_hbm.at[pl.ds(obase, CHUNK * TOP_K)])
    pltpu.sync_copy(s_v, scores_hbm.at[pl.ds(obase, CHUNK * TOP_K)])
    pltpu.sync_copy(i_v, idxs_hbm.at[pl.ds(obase, CHUNK * TOP_K)])


_router = functools.partial(
    pl.kernel,
    out_type=(
        jax.ShapeDtypeStruct((N_TOKENS * TOP_K,), jnp.float32),
        jax.ShapeDtypeStruct((N_TOKENS * TOP_K,), jnp.float32),
        jax.ShapeDtypeStruct((N_TOKENS * TOP_K,), jnp.int32),
    ),
    mesh=plsc.VectorSubcoreMesh(
        core_axis_name="c", subcore_axis_name="s",
        num_cores=NC, num_subcores=NS,
    ),
    scratch_types=[
        pltpu.VMEM((CHUNK * N_EXPERTS,), jnp.float32),
        pltpu.VMEM((N_EXPERTS,), jnp.float32),
        pltpu.VMEM((CHUNK * TOP_K,), jnp.float32),
        pltpu.VMEM((CHUNK * TOP_K,), jnp.float32),
        pltpu.VMEM((CHUNK * TOP_K,), jnp.int32),
        pltpu.SemaphoreType.DMA,
    ],
    compiler_params=pltpu.CompilerParams(
        needs_layout_passes=False,
        skip_device_barrier=True,
    ),
)(_router_body)


def _detile(o):
    return (o.reshape(N_TOKENS // BLK, TOP_K, BLK)
             .transpose(0, 2, 1)
             .reshape(N_TOKENS, TOP_K))


def kernel(x, W, b):
    logits_t = _matmul(x, W)
    gates, top_scores, top_idxs = _router(logits_t, b)
    return (_detile(gates), _detile(top_scores), _detile(top_idxs))


# speedup vs baseline: 1.0107x; 1.0107x over previous
import functools

import jax
import jax.numpy as jnp
from jax import lax
from jax.experimental import pallas as pl
from jax.experimental.pallas import tpu as pltpu
from jax.experimental.pallas import tpu_sc as plsc

N_TOKENS = 32768
D_MODEL = 2048
N_EXPERTS = 8
TOP_K = 2

NC = 2
NS = 16
L = 16
NW = NC * NS
CHUNK = N_TOKENS // NW
STEPS = CHUNK // L
BLK = 128

TOK_TILE = 1024
N_STEPS = N_TOKENS // TOK_TILE
NBUF = 4
LOOKAHEAD = 3


def _copy_slot(x_hbm, xbuf, sems, step, s):
    return pltpu.make_async_copy(
        x_hbm.at[pl.ds(step * TOK_TILE, TOK_TILE), :],
        xbuf.at[s],
        sems.at[s],
    )


def _out_copy(obuf, out_hbm, osems, step, s):
    return pltpu.make_async_copy(
        obuf.at[s],
        out_hbm.at[:, pl.ds(step * TOK_TILE, TOK_TILE)],
        osems.at[s],
    )


def _matmul_body(x_hbm, w_ref, out_hbm, xbuf, obuf, sems, osems):
    i = pl.program_id(0)

    def start(step):
        for s in range(NBUF):
            @pl.when(lax.rem(step, NBUF) == s)
            def _(s=s):
                _copy_slot(x_hbm, xbuf, sems, step, s).start()

    @pl.when(i == 0)
    def _():
        for d in range(LOOKAHEAD):
            _copy_slot(x_hbm, xbuf, sems, d, d).start()

    @pl.when(i + LOOKAHEAD < N_STEPS)
    def _():
        start(i + LOOKAHEAD)

    os_ = lax.rem(i, 2)
    for t in range(2):
        @pl.when((i >= 2) & (os_ == t))
        def _(t=t):
            _out_copy(obuf, out_hbm, osems, i - 2, t).wait()

    for s in range(NBUF):
        @pl.when(lax.rem(i, NBUF) == s)
        def _(s=s):
            _copy_slot(x_hbm, xbuf, sems, i, s).wait()
            for t in range(2):
                @pl.when(os_ == t)
                def _(s=s, t=t):
                    obuf[t] = lax.dot_general(
                        w_ref[...], xbuf[s],
                        dimension_numbers=(((0,), (1,)), ((), ())),
                        preferred_element_type=jnp.float32,
                    )
                    _out_copy(obuf, out_hbm, osems, i, t).start()

    @pl.when(i == N_STEPS - 1)
    def _():
        for t in range(2):
            @pl.when(os_ == t)
            def _(t=t):
                _out_copy(obuf, out_hbm, osems, i - 1, 1 - t).wait()
                _out_copy(obuf, out_hbm, osems, i, t).wait()


def _matmul(x, w):
    return pl.pallas_call(
        _matmul_body,
        grid=(N_STEPS,),
        in_specs=[
            pl.BlockSpec(memory_space=pl.ANY),
            pl.BlockSpec((D_MODEL, N_EXPERTS), lambda i: (0, 0)),
        ],
        out_specs=pl.BlockSpec(memory_space=pl.ANY),
        out_shape=jax.ShapeDtypeStruct((N_EXPERTS, N_TOKENS), jnp.float32),
        scratch_shapes=[
            pltpu.VMEM((NBUF, TOK_TILE, D_MODEL), jnp.float32),
            pltpu.VMEM((2, N_EXPERTS, TOK_TILE), jnp.float32),
            pltpu.SemaphoreType.DMA((NBUF,)),
            pltpu.SemaphoreType.DMA((2,)),
        ],
        compiler_params=pltpu.CompilerParams(
            dimension_semantics=("arbitrary",),
            vmem_limit_bytes=48 * 1024 * 1024,
        ),
    )(x, w)


def _router_body(logits_hbm, b_hbm, gates_hbm, scores_hbm, idxs_hbm,
                 le_v, b_v, g_v, s_v, i_v, dma_sem):
    wid = lax.axis_index("s") * NC + lax.axis_index("c")
    base = wid * CHUNK

    def le_copy(e):
        return pltpu.make_async_copy(
            logits_hbm.at[e, pl.ds(base, CHUNK)],
            le_v.at[pl.ds(e * CHUNK, CHUNK)],
            dma_sem,
        )

    for e in range(N_EXPERTS):
        le_copy(e).start()
    pltpu.sync_copy(b_hbm, b_v)
    for e in range(N_EXPERTS):
        le_copy(e).wait()
    bsp = [plsc.load_gather(b_v, [jnp.full((L,), e, jnp.int32)])
           for e in range(N_EXPERTS)]

    @plsc.parallel_loop(0, STEPS, 1, unroll=4)
    def step(j):
        off = j * L
        i1 = jnp.zeros((L,), jnp.int32)
        m1 = le_v[pl.ds(off, L)] + bsp[0]
        m2 = jnp.full((L,), -jnp.inf, jnp.float32)
        i2 = jnp.zeros((L,), jnp.int32)
        for e in range(1, N_EXPERTS):
            col = jnp.full((L,), e, jnp.int32)
            v = le_v[pl.ds(e * CHUNK + off, L)] + bsp[e]
            gt1 = v > m1
            gt2 = v > m2
            m2 = jnp.where(gt1, m1, jnp.where(gt2, v, m2))
            i2 = jnp.where(gt1, i1, jnp.where(gt2, col, i2))
            m1 = jnp.where(gt1, v, m1)
            i1 = jnp.where(gt1, col, i1)
        r = jnp.exp(m2 - m1)
        g1 = 1.0 / (1.0 + r)
        g2 = r * g1
        d0 = (j // (BLK // L)) * (TOP_K * BLK) + (j % (BLK // L)) * L
        d1 = d0 + BLK
        s_v[pl.ds(d0, L)] = m1
        s_v[pl.ds(d1, L)] = m2
        g_v[pl.ds(d0, L)] = g1
        g_v[pl.ds(d1, L)] = g2
        i_v[pl.ds(d0, L)] = i1
        i_v[pl.ds(d1, L)] = i2

    obase = base * TOP_K
    pltpu.sync_copy(g_v, gates_hbm.at[pl.ds(obase, CHUNK * TOP_K)])
    pltpu.sync_copy(s_v, scores_hbm.at[pl.ds(obase, CHUNK * TOP_K)])
    pltpu.sync_copy(i_v, idxs_hbm.at[pl.ds(obase, CHUNK * TOP_K)])


_router = functools.partial(
    pl.kernel,
    out_type=(
        jax.ShapeDtypeStruct((N_TOKENS * TOP_K,), jnp.float32),
        jax.ShapeDtypeStruct((N_TOKENS * TOP_K,), jnp.float32),
        jax.ShapeDtypeStruct((N_TOKENS * TOP_K,), jnp.int32),
    ),
    mesh=plsc.VectorSubcoreMesh(
        core_axis_name="c", subcore_axis_name="s",
        num_cores=NC, num_subcores=NS,
    ),
    scratch_types=[
        pltpu.VMEM((CHUNK * N_EXPERTS,), jnp.float32),
        pltpu.VMEM((N_EXPERTS,), jnp.float32),
        pltpu.VMEM((CHUNK * TOP_K,), jnp.float32),
        pltpu.VMEM((CHUNK * TOP_K,), jnp.float32),
        pltpu.VMEM((CHUNK * TOP_K,), jnp.int32),
        pltpu.SemaphoreType.DMA,
    ],
    compiler_params=pltpu.CompilerParams(
        needs_layout_passes=False,
        skip_device_barrier=True,
    ),
)(_router_body)


def _detile(o):
    return (o.reshape(N_TOKENS // BLK, TOP_K, BLK)
             .transpose(0, 2, 1)
             .reshape(N_TOKENS, TOP_K))


def kernel(x, W, b):
    logits_t = _matmul(x, W)
    gates, top_scores, top_idxs = _router(logits_t, b)
    return (_detile(gates), _detile(top_scores), _detile(top_idxs))


# speedup vs baseline: 1.0131x; 1.0024x over previous
import functools

import jax
import jax.numpy as jnp
from jax import lax
from jax.experimental import pallas as pl
from jax.experimental.pallas import tpu as pltpu
from jax.experimental.pallas import tpu_sc as plsc

N_TOKENS = 32768
D_MODEL = 2048
N_EXPERTS = 8
TOP_K = 2

NC = 2
NS = 16
L = 16
NW = NC * NS
CHUNK = N_TOKENS // NW
STEPS = CHUNK // L
BLK = 128

TOK_TILE = 1024
N_STEPS = N_TOKENS // TOK_TILE
NBUF = 4
LOOKAHEAD = 3


def _copy_slot(x_hbm, xbuf, sems, step, s):
    return pltpu.make_async_copy(
        x_hbm.at[pl.ds(step * TOK_TILE, TOK_TILE), :],
        xbuf.at[s],
        sems.at[s],
    )


def _out_copy(obuf, out_hbm, osems, step, s):
    return pltpu.make_async_copy(
        obuf.at[s],
        out_hbm.at[:, pl.ds(step * TOK_TILE, TOK_TILE)],
        osems.at[s],
    )


def _matmul_body(x_hbm, w_ref, out_hbm, xbuf, obuf, sems, osems):
    i = pl.program_id(0)

    def start(step):
        for s in range(NBUF):
            @pl.when(lax.rem(step, NBUF) == s)
            def _(s=s):
                _copy_slot(x_hbm, xbuf, sems, step, s).start()

    @pl.when(i == 0)
    def _():
        for d in range(LOOKAHEAD):
            _copy_slot(x_hbm, xbuf, sems, d, d).start()

    @pl.when(i + LOOKAHEAD < N_STEPS)
    def _():
        start(i + LOOKAHEAD)

    os_ = lax.rem(i, 2)
    for t in range(2):
        @pl.when((i >= 2) & (os_ == t))
        def _(t=t):
            _out_copy(obuf, out_hbm, osems, i - 2, t).wait()

    for s in range(NBUF):
        @pl.when(lax.rem(i, NBUF) == s)
        def _(s=s):
            _copy_slot(x_hbm, xbuf, sems, i, s).wait()
            for t in range(2):
                @pl.when(os_ == t)
                def _(s=s, t=t):
                    obuf[t] = lax.dot_general(
                        w_ref[...], xbuf[s],
                        dimension_numbers=(((0,), (1,)), ((), ())),
                        preferred_element_type=jnp.float32,
                    )
                    _out_copy(obuf, out_hbm, osems, i, t).start()

    @pl.when(i == N_STEPS - 1)
    def _():
        for t in range(2):
            @pl.when(os_ == t)
            def _(t=t):
                _out_copy(obuf, out_hbm, osems, i - 1, 1 - t).wait()
                _out_copy(obuf, out_hbm, osems, i, t).wait()


def _matmul(x, w):
    return pl.pallas_call(
        _matmul_body,
        grid=(N_STEPS,),
        in_specs=[
            pl.BlockSpec(memory_space=pl.ANY),
            pl.BlockSpec((D_MODEL, N_EXPERTS), lambda i: (0, 0)),
        ],
        out_specs=pl.BlockSpec(memory_space=pl.ANY),
        out_shape=jax.ShapeDtypeStruct((N_EXPERTS, N_TOKENS), jnp.float32),
        scratch_shapes=[
            pltpu.VMEM((NBUF, TOK_TILE, D_MODEL), jnp.float32),
            pltpu.VMEM((2, N_EXPERTS, TOK_TILE), jnp.float32),
            pltpu.SemaphoreType.DMA((NBUF,)),
            pltpu.SemaphoreType.DMA((2,)),
        ],
        compiler_params=pltpu.CompilerParams(
            dimension_semantics=("arbitrary",),
            vmem_limit_bytes=48 * 1024 * 1024,
        ),
    )(x, w)


def _router_body(logits_hbm, b_hbm, gates_hbm, scores_hbm, idxs_hbm,
                 le_v, b_v, g_v, s_v, i_v, dma_sem):
    wid = lax.axis_index("s") * NC + lax.axis_index("c")
    base = wid * CHUNK

    def le_copy(e):
        return pltpu.make_async_copy(
            logits_hbm.at[e, pl.ds(base, CHUNK)],
            le_v.at[pl.ds(e * CHUNK, CHUNK)],
            dma_sem,
        )

    for e in range(N_EXPERTS):
        le_copy(e).start()
    pltpu.sync_copy(b_hbm, b_v)
    for e in range(N_EXPERTS):
        le_copy(e).wait()
    bsp = [plsc.load_gather(b_v, [jnp.full((L,), e, jnp.int32)])
           for e in range(N_EXPERTS)]

    @plsc.parallel_loop(0, STEPS, 1, unroll=4)
    def step(j):
        off = j * L
        i1 = jnp.zeros((L,), jnp.int32)
        m1 = le_v[pl.ds(off, L)] + bsp[0]
        m2 = jnp.full((L,), -jnp.inf, jnp.float32)
        i2 = jnp.zeros((L,), jnp.int32)
        for e in range(1, N_EXPERTS):
            col = jnp.full((L,), e, jnp.int32)
            v = le_v[pl.ds(e * CHUNK + off, L)] + bsp[e]
            gt1 = v > m1
            gt2 = v > m2
            m2 = jnp.where(gt1, m1, jnp.where(gt2, v, m2))
            i2 = jnp.where(gt1, i1, jnp.where(gt2, col, i2))
            m1 = jnp.where(gt1, v, m1)
            i1 = jnp.where(gt1, col, i1)
        r = jnp.exp(m2 - m1)
        g1 = 1.0 / (1.0 + r)
        g2 = r * g1
        d0 = (j // (BLK // L)) * (TOP_K * BLK) + (j % (BLK // L)) * L
        d1 = d0 + BLK
        s_v[pl.ds(d0, L)] = m1
        s_v[pl.ds(d1, L)] = m2
        g_v[pl.ds(d0, L)] = g1
        g_v[pl.ds(d1, L)] = g2
        i_v[pl.ds(d0, L)] = i1
        i_v[pl.ds(d1, L)] = i2

    obase = base * TOP_K
    pltpu.sync_copy(g_v, gates_hbm.at[pl.ds(obase, CHUNK * TOP_K)])
    pltpu.sync_copy(s_v, scores_hbm.at[pl.ds(obase, CHUNK * TOP_K)])
    pltpu.sync_copy(i_v, idxs_hbm.at[pl.ds(obase, CHUNK * TOP_K)])


_router = functools.partial(
    pl.kernel,
    out_type=(
        jax.ShapeDtypeStruct((N_TOKENS * TOP_K,), jnp.float32),
        jax.ShapeDtypeStruct((N_TOKENS * TOP_K,), jnp.float32),
        jax.ShapeDtypeStruct((N_TOKENS * TOP_K,), jnp.int32),
    ),
    mesh=plsc.VectorSubcoreMesh(
        core_axis_name="c", subcore_axis_name="s",
        num_cores=NC, num_subcores=NS,
    ),
    scratch_types=[
        pltpu.VMEM((CHUNK * N_EXPERTS,), jnp.float32),
        pltpu.VMEM((N_EXPERTS,), jnp.float32),
        pltpu.VMEM((CHUNK * TOP_K,), jnp.float32),
        pltpu.VMEM((CHUNK * TOP_K,), jnp.float32),
        pltpu.VMEM((CHUNK * TOP_K,), jnp.int32),
        pltpu.SemaphoreType.DMA,
    ],
    compiler_params=pltpu.CompilerParams(needs_layout_passes=False),
)(_router_body)


def _detile(o):
    return (o.reshape(N_TOKENS // BLK, TOP_K, BLK)
             .transpose(0, 2, 1)
             .reshape(N_TOKENS, TOP_K))


def kernel(x, W, b):
    logits_t = _matmul(x, W)
    gates, top_scores, top_idxs = _router(logits_t, b)
    return (_detile(gates), _detile(top_scores), _detile(top_idxs))
